# baseline (speedup 1.0000x reference)
import functools

import jax
import jax.numpy as jnp
from jax import lax
from jax.experimental import pallas as pl
from jax.experimental.pallas import tpu as pltpu
from jax.experimental.pallas import tpu_sc as plsc

NC, NS = 2, 16
NW = NC * NS
CH = 128


def _sc_mesh():
    return plsc.VectorSubcoreMesh(core_axis_name="c", subcore_axis_name="s",
                                  num_cores=NC)


@functools.cache
def _seg_call(npad, kpw, d):
    rpt = npad // NS
    kh = kpw // 2
    assert kpw % 2 == 0 and kh >= 2

    @functools.partial(
        pl.kernel,
        out_type=jax.ShapeDtypeStruct((NC, npad, d), jnp.float32),
        mesh=_sc_mesh(),
        scratch_types=[
            pltpu.VMEM((kh, CH), jnp.int32),
            pltpu.VMEM((kh, CH), jnp.int32),
            pltpu.VMEM((2, CH, d), jnp.float32),
            pltpu.VMEM_SHARED((npad, d), jnp.float32),
            pltpu.SemaphoreType.DMA,
            pltpu.SemaphoreType.DMA,
            pltpu.SemaphoreType.DMA,
            pltpu.SemaphoreType.DMA,
        ],
    )
    def seg(p_hbm, srcs_hbm, dsts_hbm, out_hbm, src_v, dst_v, rows_v, acc,
            gsem0, gsem1, ssem0, ssem1):
        cid = lax.axis_index("c")
        sid = lax.axis_index("s")
        wid = sid * NC + cid
        z16 = jnp.zeros((16,), jnp.float32)
        gsem = (gsem0, gsem1)
        ssem = (ssem0, ssem1)

        def zrow(r, carry):
            for j0 in range(d // 16):
                rows_v[0, r, pl.ds(j0 * 16, 16)] = z16
            return carry

        lax.fori_loop(0, CH, zrow, 0)
        for k in range(rpt // CH):
            pltpu.sync_copy(rows_v.at[0], acc.at[pl.ds(sid * rpt + k * CH, CH)])
        plsc.subcore_barrier()

        for half in range(2):
            pltpu.sync_copy(srcs_hbm.at[wid, pl.ds(half * kh, kh)], src_v)
            pltpu.sync_copy(dsts_hbm.at[wid, pl.ds(half * kh, kh)], dst_v)

            pltpu.async_copy(p_hbm.at[src_v.at[0]], rows_v.at[0], gsem[0])
            pltpu.async_copy(p_hbm.at[src_v.at[1]], rows_v.at[1], gsem[1])

            def pair(j2, carry):
                j = j2 * 2
                pltpu.make_async_copy(p_hbm, rows_v.at[0], gsem[0]).wait()
                pltpu.async_copy(rows_v.at[0], acc.at[dst_v.at[j]], ssem[0],
                                 add=True)
                pltpu.make_async_copy(p_hbm, rows_v.at[1], gsem[1]).wait()
                pltpu.async_copy(rows_v.at[1], acc.at[dst_v.at[j + 1]], ssem[1],
                                 add=True)
                pltpu.make_async_copy(p_hbm, rows_v.at[0], ssem[0]).wait()
                pltpu.async_copy(p_hbm.at[src_v.at[j + 2]], rows_v.at[0],
                                 gsem[0])
                pltpu.make_async_copy(p_hbm, rows_v.at[1], ssem[1]).wait()
                pltpu.async_copy(p_hbm.at[src_v.at[j + 3]], rows_v.at[1],
                                 gsem[1])
                return carry

            lax.fori_loop(0, kh // 2 - 1, pair, 0)

            pltpu.make_async_copy(p_hbm, rows_v.at[0], gsem[0]).wait()
            pltpu.async_copy(rows_v.at[0], acc.at[dst_v.at[kh - 2]], ssem[0],
                             add=True)
            pltpu.make_async_copy(p_hbm, rows_v.at[1], gsem[1]).wait()
            pltpu.async_copy(rows_v.at[1], acc.at[dst_v.at[kh - 1]], ssem[1],
                             add=True)
            pltpu.make_async_copy(p_hbm, rows_v.at[0], ssem[0]).wait()
            pltpu.make_async_copy(p_hbm, rows_v.at[1], ssem[1]).wait()

        plsc.subcore_barrier()
        pltpu.sync_copy(acc.at[pl.ds(sid * rpt, rpt)],
                        out_hbm.at[cid, pl.ds(sid * rpt, rpt)])

    return seg


@functools.cache
def _deg_call(npad, kpw):
    rpt = npad // NS

    @functools.partial(
        pl.kernel,
        out_type=jax.ShapeDtypeStruct((NC, npad, 128), jnp.float32),
        mesh=_sc_mesh(),
        scratch_types=[
            pltpu.VMEM((kpw, CH), jnp.int32),
            pltpu.VMEM((CH, 128), jnp.float32),
            pltpu.VMEM_SHARED((npad, 128), jnp.float32),
        ],
    )
    def degk(dsts_hbm, out_hbm, dst_v, ones_v, acc):
        cid = lax.axis_index("c")
        sid = lax.axis_index("s")
        wid = sid * NC + cid

        def fill(val):
            def row(r, carry):
                for j0 in range(8):
                    ones_v[r, pl.ds(j0 * 16, 16)] = val
                return carry
            lax.fori_loop(0, CH, row, 0)

        fill(jnp.zeros((16,), jnp.float32))
        for k in range(rpt // CH):
            pltpu.sync_copy(ones_v, acc.at[pl.ds(sid * rpt + k * CH, CH)])
        fill(jnp.ones((16,), jnp.float32))
        plsc.subcore_barrier()

        pltpu.sync_copy(dsts_hbm.at[wid], dst_v)

        def chunk(j, carry):
            pltpu.sync_copy(ones_v, acc.at[dst_v.at[j]], add=True)
            return carry

        lax.fori_loop(0, kpw, chunk, 0)

        plsc.subcore_barrier()
        pltpu.sync_copy(acc.at[pl.ds(sid * rpt, rpt)],
                        out_hbm.at[cid, pl.ds(sid * rpt, rpt)])

    return degk


@functools.cache
def _mm_call(n, d, o, bs):
    def body(x_ref, w_ref, o_ref):
        o_ref[...] = jnp.dot(x_ref[...], w_ref[...],
                             preferred_element_type=jnp.float32)

    return pl.pallas_call(
        body,
        grid=(n // bs,),
        in_specs=[
            pl.BlockSpec((bs, d), lambda i: (i, 0)),
            pl.BlockSpec((d, o), lambda i: (0, 0)),
        ],
        out_specs=pl.BlockSpec((bs, o), lambda i: (i, 0)),
        out_shape=jax.ShapeDtypeStruct((n, o), jnp.float32),
    )


@functools.cache
def _post_call(n, npad, d, o, bs, last):
    def body(s_ref, deg_ref, h_ref, wr_ref, b_ref, *rest):
        if last:
            (h_out,) = rest
        else:
            wl_ref, h_out, p_out = rest
        s2 = s_ref[0] + s_ref[1]
        dg = deg_ref[0, :, 0:1] + deg_ref[1, :, 0:1]
        mean = s2 / jnp.maximum(dg, 1.0)
        z = mean + jnp.dot(h_ref[...], wr_ref[...],
                           preferred_element_type=jnp.float32) + b_ref[...]
        hn = jnp.maximum(z, 0.0)
        h_out[...] = hn
        if not last:
            p_out[...] = jnp.dot(hn, wl_ref[...],
                                 preferred_element_type=jnp.float32)

    in_specs = [
        pl.BlockSpec((NC, bs, o), lambda i: (0, i, 0)),
        pl.BlockSpec((NC, bs, 128), lambda i: (0, i, 0)),
        pl.BlockSpec((bs, d), lambda i: (i, 0)),
        pl.BlockSpec((d, o), lambda i: (0, 0)),
        pl.BlockSpec((1, o), lambda i: (0, 0)),
    ]
    out_specs = [pl.BlockSpec((bs, o), lambda i: (i, 0))]
    out_shape = [jax.ShapeDtypeStruct((n, o), jnp.float32)]
    if not last:
        in_specs.append(pl.BlockSpec((o, o), lambda i: (0, 0)))
        out_specs.append(pl.BlockSpec((bs, o), lambda i: (i, 0)))
        out_shape.append(jax.ShapeDtypeStruct((n, o), jnp.float32))

    return pl.pallas_call(
        body,
        grid=(n // bs,),
        in_specs=in_specs,
        out_specs=out_specs if not last else out_specs[0],
        out_shape=out_shape if not last else out_shape[0],
    )


def kernel(x, edge_index, Wl1, Wr1, b1, Wl2, Wr2, b2, Wl3, Wr3, b3):
    n, d = x.shape
    o = Wl1.shape[1]
    e = edge_index.shape[1]

    kpw = -(-(-(-e // (NW * CH))) // 8) * 8
    epad = NW * kpw * CH
    npad = -(-(n + 1) // (NS * CH)) * NS * CH

    src = edge_index[0]
    dst = edge_index[1]
    pad = epad - e
    if pad:
        src = jnp.concatenate([src, jnp.zeros((pad,), jnp.int32)])
        dst = jnp.concatenate([dst, jnp.full((pad,), n, jnp.int32)])
    srcs = src.reshape(NW, kpw, CH)
    dsts = dst.reshape(NW, kpw, CH)

    bs = 1000
    deg = _deg_call(npad, kpw)(dsts)

    h = x
    p = _mm_call(n, d, o, bs)(x, Wl1)
    for (Wr, b, Wlnext) in ((Wr1, b1, Wl2), (Wr2, b2, Wl3), (Wr3, b3, None)):
        s = _seg_call(npad, kpw, o)(p, srcs, dsts)
        if Wlnext is None:
            h = _post_call(n, npad, h.shape[1], o, bs, True)(
                s, deg, h, Wr, b.reshape(1, o))
        else:
            h, p = _post_call(n, npad, h.shape[1], o, bs, False)(
                s, deg, h, Wr, b.reshape(1, o), Wlnext)
    return h


# speedup vs baseline: 1.0139x; 1.0139x over previous
import functools

import jax
import jax.numpy as jnp
from jax import lax
from jax.experimental import pallas as pl
from jax.experimental.pallas import tpu as pltpu
from jax.experimental.pallas import tpu_sc as plsc

NC, NS = 2, 16
NW = NC * NS
CH = 128


def _sc_mesh():
    return plsc.VectorSubcoreMesh(core_axis_name="c", subcore_axis_name="s",
                                  num_cores=NC)


@functools.cache
def _seg_call(npad, kpw, d):
    rpt = npad // NS
    kh = kpw // 2
    assert kpw % 2 == 0 and kh >= 2

    @functools.partial(
        pl.kernel,
        out_type=jax.ShapeDtypeStruct((NC, npad, d), jnp.float32),
        mesh=_sc_mesh(),
        scratch_types=[
            pltpu.VMEM((kh, CH), jnp.int32),
            pltpu.VMEM((kh, CH), jnp.int32),
            pltpu.VMEM((2, CH, d), jnp.float32),
            pltpu.VMEM_SHARED((npad, d), jnp.float32),
            pltpu.SemaphoreType.DMA,
            pltpu.SemaphoreType.DMA,
        ],
    )
    def seg(p_hbm, srcs_hbm, dsts_hbm, out_hbm, src_v, dst_v, rows_v, acc,
            gsem0, gsem1):
        cid = lax.axis_index("c")
        sid = lax.axis_index("s")
        wid = sid * NC + cid
        z16 = jnp.zeros((16,), jnp.float32)
        gsem = (gsem0, gsem1)

        def zrow(r, carry):
            for j0 in range(d // 16):
                rows_v[0, r, pl.ds(j0 * 16, 16)] = z16
            return carry

        lax.fori_loop(0, CH, zrow, 0)
        for k in range(rpt // CH):
            pltpu.sync_copy(rows_v.at[0], acc.at[pl.ds(sid * rpt + k * CH, CH)])
        plsc.subcore_barrier()

        for half in range(2):
            pltpu.sync_copy(srcs_hbm.at[wid, pl.ds(half * kh, kh)], src_v)
            pltpu.sync_copy(dsts_hbm.at[wid, pl.ds(half * kh, kh)], dst_v)

            pltpu.async_copy(p_hbm.at[src_v.at[0]], rows_v.at[0], gsem[0])

            def pair(j2, carry):
                j = j2 * 2
                pltpu.async_copy(p_hbm.at[src_v.at[j + 1]], rows_v.at[1],
                                 gsem[1])
                pltpu.make_async_copy(p_hbm, rows_v.at[0], gsem[0]).wait()
                pltpu.sync_copy(rows_v.at[0], acc.at[dst_v.at[j]], add=True)
                pltpu.async_copy(p_hbm.at[src_v.at[j + 2]], rows_v.at[0],
                                 gsem[0])
                pltpu.make_async_copy(p_hbm, rows_v.at[1], gsem[1]).wait()
                pltpu.sync_copy(rows_v.at[1], acc.at[dst_v.at[j + 1]], add=True)
                return carry

            lax.fori_loop(0, kh // 2 - 1, pair, 0)

            pltpu.async_copy(p_hbm.at[src_v.at[kh - 1]], rows_v.at[1], gsem[1])
            pltpu.make_async_copy(p_hbm, rows_v.at[0], gsem[0]).wait()
            pltpu.sync_copy(rows_v.at[0], acc.at[dst_v.at[kh - 2]], add=True)
            pltpu.make_async_copy(p_hbm, rows_v.at[1], gsem[1]).wait()
            pltpu.sync_copy(rows_v.at[1], acc.at[dst_v.at[kh - 1]], add=True)

        plsc.subcore_barrier()
        pltpu.sync_copy(acc.at[pl.ds(sid * rpt, rpt)],
                        out_hbm.at[cid, pl.ds(sid * rpt, rpt)])

    return seg


@functools.cache
def _deg_call(npad, kpw):
    rpt = npad // NS

    @functools.partial(
        pl.kernel,
        out_type=jax.ShapeDtypeStruct((NC, npad, 128), jnp.float32),
        mesh=_sc_mesh(),
        scratch_types=[
            pltpu.VMEM((kpw, CH), jnp.int32),
            pltpu.VMEM((CH, 128), jnp.float32),
            pltpu.VMEM_SHARED((npad, 128), jnp.float32),
        ],
    )
    def degk(dsts_hbm, out_hbm, dst_v, ones_v, acc):
        cid = lax.axis_index("c")
        sid = lax.axis_index("s")
        wid = sid * NC + cid

        def fill(val):
            def row(r, carry):
                for j0 in range(8):
                    ones_v[r, pl.ds(j0 * 16, 16)] = val
                return carry
            lax.fori_loop(0, CH, row, 0)

        fill(jnp.zeros((16,), jnp.float32))
        for k in range(rpt // CH):
            pltpu.sync_copy(ones_v, acc.at[pl.ds(sid * rpt + k * CH, CH)])
        fill(jnp.ones((16,), jnp.float32))
        plsc.subcore_barrier()

        pltpu.sync_copy(dsts_hbm.at[wid], dst_v)

        def chunk(j, carry):
            pltpu.sync_copy(ones_v, acc.at[dst_v.at[j]], add=True)
            return carry

        lax.fori_loop(0, kpw, chunk, 0)

        plsc.subcore_barrier()
        pltpu.sync_copy(acc.at[pl.ds(sid * rpt, rpt)],
                        out_hbm.at[cid, pl.ds(sid * rpt, rpt)])

    return degk


@functools.cache
def _mm_call(n, d, o, bs):
    def body(x_ref, w_ref, o_ref):
        o_ref[...] = jnp.dot(x_ref[...], w_ref[...],
                             preferred_element_type=jnp.float32)

    return pl.pallas_call(
        body,
        grid=(n // bs,),
        in_specs=[
            pl.BlockSpec((bs, d), lambda i: (i, 0)),
            pl.BlockSpec((d, o), lambda i: (0, 0)),
        ],
        out_specs=pl.BlockSpec((bs, o), lambda i: (i, 0)),
        out_shape=jax.ShapeDtypeStruct((n, o), jnp.float32),
    )


@functools.cache
def _post_call(n, npad, d, o, bs, last):
    def body(s_ref, deg_ref, h_ref, wr_ref, b_ref, *rest):
        if last:
            (h_out,) = rest
        else:
            wl_ref, h_out, p_out = rest
        s2 = s_ref[0] + s_ref[1]
        dg = deg_ref[0, :, 0:1] + deg_ref[1, :, 0:1]
        mean = s2 / jnp.maximum(dg, 1.0)
        z = mean + jnp.dot(h_ref[...], wr_ref[...],
                           preferred_element_type=jnp.float32) + b_ref[...]
        hn = jnp.maximum(z, 0.0)
        h_out[...] = hn
        if not last:
            p_out[...] = jnp.dot(hn, wl_ref[...],
                                 preferred_element_type=jnp.float32)

    in_specs = [
        pl.BlockSpec((NC, bs, o), lambda i: (0, i, 0)),
        pl.BlockSpec((NC, bs, 128), lambda i: (0, i, 0)),
        pl.BlockSpec((bs, d), lambda i: (i, 0)),
        pl.BlockSpec((d, o), lambda i: (0, 0)),
        pl.BlockSpec((1, o), lambda i: (0, 0)),
    ]
    out_specs = [pl.BlockSpec((bs, o), lambda i: (i, 0))]
    out_shape = [jax.ShapeDtypeStruct((n, o), jnp.float32)]
    if not last:
        in_specs.append(pl.BlockSpec((o, o), lambda i: (0, 0)))
        out_specs.append(pl.BlockSpec((bs, o), lambda i: (i, 0)))
        out_shape.append(jax.ShapeDtypeStruct((n, o), jnp.float32))

    return pl.pallas_call(
        body,
        grid=(n // bs,),
        in_specs=in_specs,
        out_specs=out_specs if not last else out_specs[0],
        out_shape=out_shape if not last else out_shape[0],
    )


def kernel(x, edge_index, Wl1, Wr1, b1, Wl2, Wr2, b2, Wl3, Wr3, b3):
    n, d = x.shape
    o = Wl1.shape[1]
    e = edge_index.shape[1]

    kpw = -(-(-(-e // (NW * CH))) // 8) * 8
    epad = NW * kpw * CH
    npad = -(-(n + 1) // (NS * CH)) * NS * CH

    src = edge_index[0]
    dst = edge_index[1]
    pad = epad - e
    if pad:
        src = jnp.concatenate([src, jnp.zeros((pad,), jnp.int32)])
        dst = jnp.concatenate([dst, jnp.full((pad,), n, jnp.int32)])
    srcs = src.reshape(NW, kpw, CH)
    dsts = dst.reshape(NW, kpw, CH)

    bs = 1000
    deg = _deg_call(npad, kpw)(dsts)

    h = x
    p = _mm_call(n, d, o, bs)(x, Wl1)
    for (Wr, b, Wlnext) in ((Wr1, b1, Wl2), (Wr2, b2, Wl3), (Wr3, b3, None)):
        s = _seg_call(npad, kpw, o)(p, srcs, dsts)
        if Wlnext is None:
            h = _post_call(n, npad, h.shape[1], o, bs, True)(
                s, deg, h, Wr, b.reshape(1, o))
        else:
            h, p = _post_call(n, npad, h.shape[1], o, bs, False)(
                s, deg, h, Wr, b.reshape(1, o), Wlnext)
    return h
